# baseline (speedup 1.0000x reference)
import functools

import jax
import jax.numpy as jnp
from jax import lax
from jax.experimental import pallas as pl
from jax.experimental.pallas import tpu as pltpu
from jax.experimental.pallas import tpu_sc as plsc

_T = 0.07
_TOP_K = 64
_NUM_CLASSES = 100
_K_QUEUE = 1000
_G = 128


def _head_body(x_ref, w1_ref, b1_ref, w2_ref, b2_ref, logits_ref, qproj_ref):
    x = x_ref[...]
    h = jnp.dot(x, w1_ref[...], preferred_element_type=jnp.float32)
    h = jnp.maximum(h + b1_ref[...], 0.0)
    logits_ref[...] = jnp.dot(h, w2_ref[...], preferred_element_type=jnp.float32) + b2_ref[...]
    nrm = jnp.sum(x * x, axis=1, keepdims=True)
    qproj_ref[...] = x * jax.lax.rsqrt(nrm)


def _sim_body(y_ref, q_ref, f_ref, cos_ref, cmax_ref, pos_ref, *, n_total):
    j = pl.program_id(0)
    s = jax.lax.dot_general(q_ref[...], f_ref[...], (((1,), (1,)), ((), ())),
                            preferred_element_type=jnp.float32)
    cols = j * _G + jax.lax.broadcasted_iota(jnp.int32, s.shape, 1)
    valid = cols < n_total
    cls = cols // _K_QUEUE
    pos = jnp.logical_and(cls == y_ref[...], valid)
    neg_inf = jnp.float32(-jnp.inf)
    negval = jnp.where(jnp.logical_or(pos, jnp.logical_not(valid)), neg_inf, s)
    posval = jnp.where(pos, s, neg_inf)
    cos_ref[...] = negval[None]
    cmax_ref[...] = jnp.max(negval, axis=1, keepdims=True)[None]
    pmax = jnp.max(posval, axis=1, keepdims=True)

    @pl.when(j == 0)
    def _():
        pos_ref[...] = pmax

    @pl.when(j > 0)
    def _():
        pos_ref[...] = jnp.maximum(pos_ref[...], pmax)


def _lane_iota(shape):
    return jax.lax.broadcasted_iota(jnp.int32, shape, len(shape) - 1)


def _xor_shuffle(x, j):
    L = x.shape[-1]
    xl = jnp.concatenate([x[:, j:], x[:, :j]], axis=1)
    xr = jnp.concatenate([x[:, L - j:], x[:, :L - j]], axis=1)
    return jnp.where((_lane_iota(x.shape) & j) == 0, xl, xr)


def _ce(v, idx, j, keep_max):
    pv = _xor_shuffle(v, j)
    nv = jnp.where(keep_max, jnp.maximum(v, pv), jnp.minimum(v, pv))
    if idx is None:
        return nv, None
    pidx = _xor_shuffle(idx, j)
    take_p = (keep_max & (pv > v)) | (jnp.logical_not(keep_max) & (pv < v))
    return nv, jnp.where(take_p, pidx, idx)


def _merge64(v, idx, desc):
    i = _lane_iota(v.shape)
    for j in (32, 16, 8, 4, 2, 1):
        keep_max = jnp.logical_not(((i & j) == 0) ^ desc)
        v, idx = _ce(v, idx, j, keep_max)
    return v, idx


def _sort64(v, idx, desc):
    i = _lane_iota(v.shape)
    for k in (2, 4, 8, 16, 32):
        desc_k = ((i & k) != 0) ^ desc
        j = k // 2
        while j:
            keep_max = jnp.logical_not(((i & j) == 0) ^ desc_k)
            v, idx = _ce(v, idx, j, keep_max)
            j //= 2
    return _merge64(v, idx, desc)


def _top64(v, idx):
    W = v.shape[-1]
    desc = (_lane_iota(v.shape) & (W // 2)) == 0
    v, idx = _sort64(v, idx, desc)
    H = W // 2
    while H >= 64:
        a, b = v[:, :H], v[:, H:]
        if idx is not None:
            idx = jnp.where(b > a, idx[:, H:], idx[:, :H])
        v = jnp.maximum(a, b)
        i = _lane_iota(v.shape)
        desc = ((i & (H // 2)) == 0) if H > 64 else (i >= 0)
        v, idx = _merge64(v, idx, desc)
        H //= 2
    return v, idx


def _select_body(cmax_ref, idx_ref):
    v = cmax_ref[...]
    R, n = v.shape
    W = 1 << (n - 1).bit_length()
    if W > n:
        v = jnp.concatenate(
            [v, jnp.full((R, W - n), -jnp.inf, jnp.float32)], axis=1)
    _, idx = _top64(v, _lane_iota(v.shape))
    idx_ref[...] = idx


def _gather_body(table_hbm, idx_hbm, pool_hbm, idx_v, gidx_v, row_v, sem,
                 *, rows_per_w, b_total, n_cores):
    wid = lax.axis_index("s") * n_cores + lax.axis_index("c")
    base = wid * rows_per_w
    pltpu.sync_copy(idx_hbm.at[pl.ds(base * _TOP_K, rows_per_w * _TOP_K)], idx_v)
    for r in range(rows_per_w):
        row = base + r
        for k in range(_TOP_K // 16):
            gidx_v[pl.ds(k * 16, 16)] = idx_v[pl.ds(r * _TOP_K + k * 16, 16)] * b_total + row
        pltpu.async_copy(table_hbm.at[gidx_v], row_v, sem).wait()
        pltpu.sync_copy(row_v, pool_hbm.at[row])


def _final_body(pool_ref, pos_ref, out_ref):
    neg, _ = _top64(pool_ref[...], None)
    out_ref[...] = jnp.concatenate([pos_ref[...], neg], axis=1) / _T


def kernel(x_q, y_batch, W1, b1, W2, b2, feature_queues):
    B, D = x_q.shape
    C = W2.shape[1]
    n_total = feature_queues.shape[0] * feature_queues.shape[1]
    all_features = feature_queues.reshape(n_total, D)
    n_chunks = (n_total + _G - 1) // _G

    logits, q_proj = pl.pallas_call(
        _head_body,
        out_shape=(jax.ShapeDtypeStruct((B, C), jnp.float32),
                   jax.ShapeDtypeStruct((B, D), jnp.float32)),
    )(x_q, W1, b1[None, :], W2, b2[None, :])

    cos, cmax, pos_score = pl.pallas_call(
        functools.partial(_sim_body, n_total=n_total),
        grid=(n_chunks,),
        in_specs=[
            pl.BlockSpec((B, 1), lambda j: (0, 0)),
            pl.BlockSpec((B, D), lambda j: (0, 0)),
            pl.BlockSpec((_G, D), lambda j: (j, 0)),
        ],
        out_specs=[
            pl.BlockSpec((1, B, _G), lambda j: (j, 0, 0)),
            pl.BlockSpec((1, B, 1), lambda j: (j, 0, 0)),
            pl.BlockSpec((B, 1), lambda j: (0, 0)),
        ],
        out_shape=[
            jax.ShapeDtypeStruct((n_chunks, B, _G), jnp.float32),
            jax.ShapeDtypeStruct((n_chunks, B, 1), jnp.float32),
            jax.ShapeDtypeStruct((B, 1), jnp.float32),
        ],
    )(y_batch.astype(jnp.int32)[:, None], q_proj, all_features)

    cmax = cmax.reshape(n_chunks, B).T
    RB = 256
    chunk_idx = pl.pallas_call(
        _select_body,
        grid=(B // RB,),
        in_specs=[pl.BlockSpec((RB, n_chunks), lambda j: (j, 0))],
        out_specs=pl.BlockSpec((RB, _TOP_K), lambda j: (j, 0)),
        out_shape=jax.ShapeDtypeStruct((B, _TOP_K), jnp.int32),
    )(cmax)

    info = plsc.get_sparse_core_info()
    n_workers = info.num_cores * info.num_subcores
    rows_per_w = B // n_workers
    pool = pl.kernel(
        functools.partial(_gather_body, rows_per_w=rows_per_w,
                          b_total=B, n_cores=info.num_cores),
        out_type=jax.ShapeDtypeStruct((B, _TOP_K, _G), jnp.float32),
        scratch_types=[
            pltpu.VMEM((rows_per_w * _TOP_K,), jnp.int32),
            pltpu.VMEM((_TOP_K,), jnp.int32),
            pltpu.VMEM((_TOP_K, _G), jnp.float32),
            pltpu.SemaphoreType.DMA,
        ],
        mesh=plsc.VectorSubcoreMesh(core_axis_name="c", subcore_axis_name="s"),
    )(cos.reshape(n_chunks * B, _G), chunk_idx.reshape(B * _TOP_K))

    logits_con = pl.pallas_call(
        _final_body,
        grid=(B // RB,),
        in_specs=[pl.BlockSpec((RB, _TOP_K * _G), lambda j: (j, 0)),
                  pl.BlockSpec((RB, 1), lambda j: (j, 0))],
        out_specs=pl.BlockSpec((RB, 1 + _TOP_K), lambda j: (j, 0)),
        out_shape=jax.ShapeDtypeStruct((B, 1 + _TOP_K), jnp.float32),
    )(pool.reshape(B, _TOP_K * _G), pos_score)
    return (logits, logits_con)


# speedup vs baseline: 1.2695x; 1.2695x over previous
import functools

import jax
import jax.numpy as jnp
from jax import lax
from jax.experimental import pallas as pl
from jax.experimental.pallas import tpu as pltpu
from jax.experimental.pallas import tpu_sc as plsc

_T = 0.07
_TOP_K = 64
_NUM_CLASSES = 100
_K_QUEUE = 1000
_G = 128
_TILE = 512
_CPT = _TILE // _G


def _head_body(x_ref, w1_ref, b1_ref, w2_ref, b2_ref, logits_ref, qproj_ref):
    x = x_ref[...]
    h = jnp.dot(x, w1_ref[...], preferred_element_type=jnp.float32)
    h = jnp.maximum(h + b1_ref[...], 0.0)
    logits_ref[...] = jnp.dot(h, w2_ref[...], preferred_element_type=jnp.float32) + b2_ref[...]
    nrm = jnp.sum(x * x, axis=1, keepdims=True)
    qproj_ref[...] = x * jax.lax.rsqrt(nrm)


def _sim_body(y_ref, q_ref, f_ref, cos_ref, cmax_ref, pos_ref, *, n_total):
    j = pl.program_id(0)
    s = jax.lax.dot_general(q_ref[...], f_ref[...], (((1,), (1,)), ((), ())),
                            preferred_element_type=jnp.float32)
    cols = j * _TILE + jax.lax.broadcasted_iota(jnp.int32, s.shape, 1)
    valid = cols < n_total
    cls = cols // _K_QUEUE
    pos = jnp.logical_and(cls == y_ref[...], valid)
    neg_inf = jnp.float32(-jnp.inf)
    negval = jnp.where(jnp.logical_or(pos, jnp.logical_not(valid)), neg_inf, s)
    posval = jnp.where(pos, s, neg_inf)
    for c in range(_CPT):
        blk = negval[:, c * _G:(c + 1) * _G]
        cos_ref[c, :, :] = blk
        cmax_ref[c, :, :] = jnp.max(blk, axis=1, keepdims=True)
    pmax = jnp.max(posval, axis=1, keepdims=True)

    @pl.when(j == 0)
    def _():
        pos_ref[...] = pmax

    @pl.when(j > 0)
    def _():
        pos_ref[...] = jnp.maximum(pos_ref[...], pmax)


def _lane_iota(shape):
    return jax.lax.broadcasted_iota(jnp.int32, shape, len(shape) - 1)


def _xor_shuffle(x, j):
    L = x.shape[-1]
    xl = jnp.concatenate([x[:, j:], x[:, :j]], axis=1)
    xr = jnp.concatenate([x[:, L - j:], x[:, :L - j]], axis=1)
    return jnp.where((_lane_iota(x.shape) & j) == 0, xl, xr)


def _ce(v, idx, j, keep_max):
    pv = _xor_shuffle(v, j)
    nv = jnp.where(keep_max, jnp.maximum(v, pv), jnp.minimum(v, pv))
    if idx is None:
        return nv, None
    pidx = _xor_shuffle(idx, j)
    take_p = (keep_max & (pv > v)) | (jnp.logical_not(keep_max) & (pv < v))
    return nv, jnp.where(take_p, pidx, idx)


def _merge64(v, idx, desc):
    i = _lane_iota(v.shape)
    for j in (32, 16, 8, 4, 2, 1):
        keep_max = jnp.logical_not(((i & j) == 0) ^ desc)
        v, idx = _ce(v, idx, j, keep_max)
    return v, idx


def _sort64(v, idx, desc):
    i = _lane_iota(v.shape)
    for k in (2, 4, 8, 16, 32):
        desc_k = ((i & k) != 0) ^ desc
        j = k // 2
        while j:
            keep_max = jnp.logical_not(((i & j) == 0) ^ desc_k)
            v, idx = _ce(v, idx, j, keep_max)
            j //= 2
    return _merge64(v, idx, desc)


def _top64(v, idx):
    W = v.shape[-1]
    desc = (_lane_iota(v.shape) & (W // 2)) == 0
    v, idx = _sort64(v, idx, desc)
    H = W // 2
    while H >= 64:
        a, b = v[:, :H], v[:, H:]
        if idx is not None:
            idx = jnp.where(b > a, idx[:, H:], idx[:, :H])
        v = jnp.maximum(a, b)
        i = _lane_iota(v.shape)
        desc = ((i & (H // 2)) == 0) if H > 64 else (i >= 0)
        v, idx = _merge64(v, idx, desc)
        H //= 2
    return v, idx


def _select_body(cmax_ref, idx_ref):
    v = cmax_ref[...]
    R, n = v.shape
    W = 1 << (n - 1).bit_length()
    if W > n:
        v = jnp.concatenate(
            [v, jnp.full((R, W - n), -jnp.inf, jnp.float32)], axis=1)
    _, idx = _top64(v, _lane_iota(v.shape))
    idx_ref[...] = idx


def _gather_body(table_hbm, idx_hbm, pool_hbm, idx_v, gidx_v, row_v, sem,
                 *, rows_per_w, b_total, n_cores):
    wid = lax.axis_index("s") * n_cores + lax.axis_index("c")
    base = wid * rows_per_w
    pltpu.sync_copy(idx_hbm.at[pl.ds(base * _TOP_K, rows_per_w * _TOP_K)], idx_v)
    for r in range(rows_per_w):
        row = base + r
        for k in range(_TOP_K // 16):
            gidx_v[pl.ds(k * 16, 16)] = idx_v[pl.ds(r * _TOP_K + k * 16, 16)] * b_total + row
        pltpu.async_copy(table_hbm.at[gidx_v], row_v, sem).wait()
        pltpu.sync_copy(row_v, pool_hbm.at[row])


def _final_body(pool_ref, pos_ref, out_ref):
    neg, _ = _top64(pool_ref[...], None)
    out_ref[...] = jnp.concatenate([pos_ref[...], neg], axis=1) / _T


def kernel(x_q, y_batch, W1, b1, W2, b2, feature_queues):
    B, D = x_q.shape
    C = W2.shape[1]
    n_total = feature_queues.shape[0] * feature_queues.shape[1]
    all_features = feature_queues.reshape(n_total, D)
    n_tiles = (n_total + _TILE - 1) // _TILE
    n_chunks = n_tiles * _CPT

    logits, q_proj = pl.pallas_call(
        _head_body,
        out_shape=(jax.ShapeDtypeStruct((B, C), jnp.float32),
                   jax.ShapeDtypeStruct((B, D), jnp.float32)),
    )(x_q, W1, b1[None, :], W2, b2[None, :])

    cos, cmax, pos_score = pl.pallas_call(
        functools.partial(_sim_body, n_total=n_total),
        grid=(n_tiles,),
        in_specs=[
            pl.BlockSpec((B, 1), lambda j: (0, 0)),
            pl.BlockSpec((B, D), lambda j: (0, 0)),
            pl.BlockSpec((_TILE, D), lambda j: (j, 0)),
        ],
        out_specs=[
            pl.BlockSpec((_CPT, B, _G), lambda j: (j, 0, 0)),
            pl.BlockSpec((_CPT, B, 1), lambda j: (j, 0, 0)),
            pl.BlockSpec((B, 1), lambda j: (0, 0)),
        ],
        out_shape=[
            jax.ShapeDtypeStruct((n_chunks, B, _G), jnp.float32),
            jax.ShapeDtypeStruct((n_chunks, B, 1), jnp.float32),
            jax.ShapeDtypeStruct((B, 1), jnp.float32),
        ],
    )(y_batch.astype(jnp.int32)[:, None], q_proj, all_features)

    cmax = cmax.reshape(n_chunks, B).T
    RB = 256
    chunk_idx = pl.pallas_call(
        _select_body,
        grid=(B // RB,),
        in_specs=[pl.BlockSpec((RB, n_chunks), lambda j: (j, 0))],
        out_specs=pl.BlockSpec((RB, _TOP_K), lambda j: (j, 0)),
        out_shape=jax.ShapeDtypeStruct((B, _TOP_K), jnp.int32),
    )(cmax)

    info = plsc.get_sparse_core_info()
    n_workers = info.num_cores * info.num_subcores
    rows_per_w = B // n_workers
    pool = pl.kernel(
        functools.partial(_gather_body, rows_per_w=rows_per_w,
                          b_total=B, n_cores=info.num_cores),
        out_type=jax.ShapeDtypeStruct((B, _TOP_K, _G), jnp.float32),
        scratch_types=[
            pltpu.VMEM((rows_per_w * _TOP_K,), jnp.int32),
            pltpu.VMEM((_TOP_K,), jnp.int32),
            pltpu.VMEM((_TOP_K, _G), jnp.float32),
            pltpu.SemaphoreType.DMA,
        ],
        mesh=plsc.VectorSubcoreMesh(core_axis_name="c", subcore_axis_name="s"),
    )(cos.reshape(n_chunks * B, _G), chunk_idx.reshape(B * _TOP_K))

    logits_con = pl.pallas_call(
        _final_body,
        grid=(B // RB,),
        in_specs=[pl.BlockSpec((RB, _TOP_K * _G), lambda j: (j, 0)),
                  pl.BlockSpec((RB, 1), lambda j: (j, 0))],
        out_specs=pl.BlockSpec((RB, 1 + _TOP_K), lambda j: (j, 0)),
        out_shape=jax.ShapeDtypeStruct((B, 1 + _TOP_K), jnp.float32),
    )(pool.reshape(B, _TOP_K * _G), pos_score)
    return (logits, logits_con)


# speedup vs baseline: 1.2871x; 1.0139x over previous
import functools

import jax
import jax.numpy as jnp
from jax import lax
from jax.experimental import pallas as pl
from jax.experimental.pallas import tpu as pltpu
from jax.experimental.pallas import tpu_sc as plsc

_T = 0.07
_TOP_K = 64
_NUM_CLASSES = 100
_K_QUEUE = 1000
_G = 128
_TILE = 512
_CPT = _TILE // _G


def _head_body(x_ref, w1_ref, b1_ref, w2_ref, b2_ref, logits_ref, qproj_ref):
    x = x_ref[...]
    h = jnp.dot(x, w1_ref[...], preferred_element_type=jnp.float32)
    h = jnp.maximum(h + b1_ref[...], 0.0)
    logits_ref[...] = jnp.dot(h, w2_ref[...], preferred_element_type=jnp.float32) + b2_ref[...]
    nrm = jnp.sum(x * x, axis=1, keepdims=True)
    qproj_ref[...] = x * jax.lax.rsqrt(nrm)


def _sim_body(y_ref, q_ref, f_ref, cos_ref, cmax_ref, pos_ref, *, n_total):
    j = pl.program_id(0)
    s = jax.lax.dot_general(q_ref[...], f_ref[...], (((1,), (1,)), ((), ())),
                            preferred_element_type=jnp.float32)
    cols = j * _TILE + jax.lax.broadcasted_iota(jnp.int32, s.shape, 1)
    valid = cols < n_total
    cls = cols // _K_QUEUE
    pos = jnp.logical_and(cls == y_ref[...], valid)
    neg_inf = jnp.float32(-jnp.inf)
    negval = jnp.where(jnp.logical_or(pos, jnp.logical_not(valid)), neg_inf, s)
    posval = jnp.where(pos, s, neg_inf)
    for c in range(_CPT):
        blk = negval[:, c * _G:(c + 1) * _G]
        cos_ref[c, :, :] = blk
        cmax_ref[c, :, :] = jnp.max(blk, axis=1, keepdims=True)
    pmax = jnp.max(posval, axis=1, keepdims=True)

    @pl.when(j == 0)
    def _():
        pos_ref[...] = pmax

    @pl.when(j > 0)
    def _():
        pos_ref[...] = jnp.maximum(pos_ref[...], pmax)


def _lane_iota(shape):
    return jax.lax.broadcasted_iota(jnp.int32, shape, len(shape) - 1)


def _xor_shuffle(x, j):
    L = x.shape[-1]
    xl = jnp.concatenate([x[:, j:], x[:, :j]], axis=1)
    xr = jnp.concatenate([x[:, L - j:], x[:, :L - j]], axis=1)
    return jnp.where((_lane_iota(x.shape) & j) == 0, xl, xr)


def _ce(v, idx, j, keep_max):
    pv = _xor_shuffle(v, j)
    nv = jnp.where(keep_max, jnp.maximum(v, pv), jnp.minimum(v, pv))
    if idx is None:
        return nv, None
    pidx = _xor_shuffle(idx, j)
    take_p = (keep_max & (pv > v)) | (jnp.logical_not(keep_max) & (pv < v))
    return nv, jnp.where(take_p, pidx, idx)


def _merge64(v, idx, desc):
    i = _lane_iota(v.shape)
    for j in (32, 16, 8, 4, 2, 1):
        keep_max = jnp.logical_not(((i & j) == 0) ^ desc)
        v, idx = _ce(v, idx, j, keep_max)
    return v, idx


def _sort64(v, idx, desc):
    i = _lane_iota(v.shape)
    for k in (2, 4, 8, 16, 32):
        desc_k = ((i & k) != 0) ^ desc
        j = k // 2
        while j:
            keep_max = jnp.logical_not(((i & j) == 0) ^ desc_k)
            v, idx = _ce(v, idx, j, keep_max)
            j //= 2
    return _merge64(v, idx, desc)


def _top64(v, idx):
    W = v.shape[-1]
    desc = (_lane_iota(v.shape) & (W // 2)) == 0
    v, idx = _sort64(v, idx, desc)
    H = W // 2
    while H >= 64:
        a, b = v[:, :H], v[:, H:]
        if idx is not None:
            idx = jnp.where(b > a, idx[:, H:], idx[:, :H])
        v = jnp.maximum(a, b)
        i = _lane_iota(v.shape)
        desc = ((i & (H // 2)) == 0) if H > 64 else (i >= 0)
        v, idx = _merge64(v, idx, desc)
        H //= 2
    return v, idx


def _select_body(cmax_ref, idx_ref):
    v = cmax_ref[...]
    R, n = v.shape
    W = 1 << (n - 1).bit_length()
    if W > n:
        v = jnp.concatenate(
            [v, jnp.full((R, W - n), -jnp.inf, jnp.float32)], axis=1)
    _, idx = _top64(v, _lane_iota(v.shape))
    idx_ref[...] = idx


def _gather_body(table_hbm, idx_hbm, pool_hbm, idx_v, gidx_v, row_v, sem,
                 *, rows_per_w, b_total, n_cores):
    wid = lax.axis_index("s") * n_cores + lax.axis_index("c")
    base = wid * rows_per_w
    pltpu.sync_copy(idx_hbm.at[pl.ds(base * _TOP_K, rows_per_w * _TOP_K)], idx_v)
    for r in range(rows_per_w):
        row = base + r
        for k in range(_TOP_K // 16):
            gidx_v[pl.ds(k * 16, 16)] = idx_v[pl.ds(r * _TOP_K + k * 16, 16)] * b_total + row
        pltpu.async_copy(table_hbm.at[gidx_v], row_v, sem).wait()
        pltpu.sync_copy(row_v, pool_hbm.at[row])


def _sub_iota(shape):
    return jax.lax.broadcasted_iota(jnp.int32, shape, 1)


def _lane3_iota(shape):
    return jax.lax.broadcasted_iota(jnp.int32, shape, 2)


def _xor_shuffle_sub(x, j):
    S = x.shape[1]
    xl = jnp.concatenate([x[:, j:, :], x[:, :j, :]], axis=1)
    xr = jnp.concatenate([x[:, S - j:, :], x[:, :S - j, :]], axis=1)
    return jnp.where((_sub_iota(x.shape) & j) == 0, xl, xr)


def _ce_sub(v, j, keep_max):
    p = _xor_shuffle_sub(v, j)
    return jnp.where(keep_max, jnp.maximum(v, p), jnp.minimum(v, p))


def _merge64_sub(v, desc):
    s = _sub_iota(v.shape)
    for j in (32, 16, 8, 4, 2, 1):
        keep_max = jnp.logical_not(((s & j) == 0) ^ desc)
        v = _ce_sub(v, j, keep_max)
    return v


def _sort64_sub(v, desc):
    s = _sub_iota(v.shape)
    for k in (2, 4, 8, 16, 32):
        desc_k = ((s & k) != 0) ^ desc
        j = k // 2
        while j:
            keep_max = jnp.logical_not(((s & j) == 0) ^ desc_k)
            v = _ce_sub(v, j, keep_max)
            j //= 2
    return _merge64_sub(v, desc)


def _final_body(pool_ref, out_ref):
    v = pool_ref[...]
    L = v.shape[2]
    desc = (_lane3_iota(v.shape) & (L // 2)) == 0
    v = _sort64_sub(v, desc)
    H = L // 2
    while H >= 1:
        v = jnp.maximum(v[:, :, :H], v[:, :, H:])
        i = _lane3_iota(v.shape)
        desc = ((i & (H // 2)) == 0) if H > 1 else (i >= 0)
        v = _merge64_sub(v, desc)
        H //= 2
    out_ref[...] = v


def kernel(x_q, y_batch, W1, b1, W2, b2, feature_queues):
    B, D = x_q.shape
    C = W2.shape[1]
    n_total = feature_queues.shape[0] * feature_queues.shape[1]
    all_features = feature_queues.reshape(n_total, D)
    n_tiles = (n_total + _TILE - 1) // _TILE
    n_chunks = n_tiles * _CPT

    logits, q_proj = pl.pallas_call(
        _head_body,
        out_shape=(jax.ShapeDtypeStruct((B, C), jnp.float32),
                   jax.ShapeDtypeStruct((B, D), jnp.float32)),
    )(x_q, W1, b1[None, :], W2, b2[None, :])

    cos, cmax, pos_score = pl.pallas_call(
        functools.partial(_sim_body, n_total=n_total),
        grid=(n_tiles,),
        in_specs=[
            pl.BlockSpec((B, 1), lambda j: (0, 0)),
            pl.BlockSpec((B, D), lambda j: (0, 0)),
            pl.BlockSpec((_TILE, D), lambda j: (j, 0)),
        ],
        out_specs=[
            pl.BlockSpec((_CPT, B, _G), lambda j: (j, 0, 0)),
            pl.BlockSpec((_CPT, B, 1), lambda j: (j, 0, 0)),
            pl.BlockSpec((B, 1), lambda j: (0, 0)),
        ],
        out_shape=[
            jax.ShapeDtypeStruct((n_chunks, B, _G), jnp.float32),
            jax.ShapeDtypeStruct((n_chunks, B, 1), jnp.float32),
            jax.ShapeDtypeStruct((B, 1), jnp.float32),
        ],
    )(y_batch.astype(jnp.int32)[:, None], q_proj, all_features)

    cmax = cmax.reshape(n_chunks, B).T
    RB = 256
    chunk_idx = pl.pallas_call(
        _select_body,
        grid=(B // RB,),
        in_specs=[pl.BlockSpec((RB, n_chunks), lambda j: (j, 0))],
        out_specs=pl.BlockSpec((RB, _TOP_K), lambda j: (j, 0)),
        out_shape=jax.ShapeDtypeStruct((B, _TOP_K), jnp.int32),
    )(cmax)

    info = plsc.get_sparse_core_info()
    n_workers = info.num_cores * info.num_subcores
    rows_per_w = B // n_workers
    pool = pl.kernel(
        functools.partial(_gather_body, rows_per_w=rows_per_w,
                          b_total=B, n_cores=info.num_cores),
        out_type=jax.ShapeDtypeStruct((B, _TOP_K, _G), jnp.float32),
        scratch_types=[
            pltpu.VMEM((rows_per_w * _TOP_K,), jnp.int32),
            pltpu.VMEM((_TOP_K,), jnp.int32),
            pltpu.VMEM((_TOP_K, _G), jnp.float32),
            pltpu.SemaphoreType.DMA,
        ],
        mesh=plsc.VectorSubcoreMesh(core_axis_name="c", subcore_axis_name="s"),
    )(cos.reshape(n_chunks * B, _G), chunk_idx.reshape(B * _TOP_K))

    neg = pl.pallas_call(
        _final_body,
        grid=(B // RB,),
        in_specs=[pl.BlockSpec((RB, _TOP_K, _G), lambda j: (j, 0, 0))],
        out_specs=pl.BlockSpec((RB, _TOP_K, 1), lambda j: (j, 0, 0)),
        out_shape=jax.ShapeDtypeStruct((B, _TOP_K, 1), jnp.float32),
    )(pool)
    logits_con = jnp.concatenate([pos_score, neg.reshape(B, _TOP_K)], axis=1) / _T
    return (logits, logits_con)
